# Initial kernel scaffold; baseline (speedup 1.0000x reference)
import jax
import jax.numpy as jnp
from jax.experimental import pallas as pl


def kernel(x, adj, inv, W_m1, b_m1, W_m2, b_m2, W_u1, b_u1, W_u2, b_u2):
    raise NotImplementedError("write your pallas kernel here")


# speedup vs baseline: 2.5121x; 2.5121x over previous
import functools

import jax
import jax.numpy as jnp
from jax import lax
from jax.experimental import pallas as pl
from jax.experimental.pallas import tpu as pltpu
from jax.experimental.pallas import tpu_sc as plsc

N_NODES = 10000
N_EDGES = 320000
H = 128
D_INV = 16

NC = 2
NS = 16
NW = NC * NS

EPW = N_EDGES // NW
CHUNK = 80
NCHUNK = EPW // CHUNK
STRIPE = 640
ZROWS = 80

@functools.cache
def _sc_mesh():
    return plsc.VectorSubcoreMesh(
        core_axis_name="c", subcore_axis_name="s", num_cores=NC, num_subcores=NS
    )


def _silu(v):
    return v * (1.0 / (1.0 + jnp.exp(-v)))


def _pq_body(x_ref, wa_ref, wb_ref, b_ref, p_ref, q_ref):
    x = x_ref[...]
    p_ref[...] = jnp.dot(x, wa_ref[...], preferred_element_type=jnp.float32) + b_ref[...]
    q_ref[...] = jnp.dot(x, wb_ref[...], preferred_element_type=jnp.float32)


def _pq(x, wa, wb, b):
    bn = 2000
    grid = (N_NODES // bn,)
    return pl.pallas_call(
        _pq_body,
        grid=grid,
        in_specs=[
            pl.BlockSpec((bn, H), lambda i: (i, 0)),
            pl.BlockSpec((H, H), lambda i: (0, 0)),
            pl.BlockSpec((H, H), lambda i: (0, 0)),
            pl.BlockSpec((1, H), lambda i: (0, 0)),
        ],
        out_specs=[
            pl.BlockSpec((bn, H), lambda i: (i, 0)),
            pl.BlockSpec((bn, H), lambda i: (i, 0)),
        ],
        out_shape=[
            jax.ShapeDtypeStruct((N_NODES, H), jnp.float32),
            jax.ShapeDtypeStruct((N_NODES, H), jnp.float32),
        ],
    )(x, wa, wb, b)


@functools.cache
def _gather_add_kernel():
    return pl.kernel(
        _gather_add_body,
        out_type=jax.ShapeDtypeStruct((N_EDGES, H), jnp.float32),
        mesh=_sc_mesh(),
        scratch_types=[
            pltpu.VMEM((CHUNK,), jnp.int32),
            pltpu.VMEM((CHUNK,), jnp.int32),
            pltpu.VMEM((CHUNK, H), jnp.float32),
            pltpu.VMEM((CHUNK, H), jnp.float32),
            pltpu.SemaphoreType.DMA,
            pltpu.SemaphoreType.DMA,
        ],
    )


def _gather_add_body(p_hbm, q_hbm, dst_hbm, src_hbm, t_hbm, idx_d, idx_s, bufp, bufq, sem1, sem2):
    wid = lax.axis_index("s") * NC + lax.axis_index("c")
    wbase = wid * EPW

    def chunk(c, carry):
        base = pl.multiple_of(wbase + c * CHUNK, 8)
        pltpu.sync_copy(dst_hbm.at[pl.ds(base, CHUNK)], idx_d)
        pltpu.sync_copy(src_hbm.at[pl.ds(base, CHUNK)], idx_s)
        cp1 = pltpu.async_copy(p_hbm.at[idx_d], bufp, sem1)
        cp2 = pltpu.async_copy(q_hbm.at[idx_s], bufq, sem2)
        cp1.wait()
        cp2.wait()

        def addrow(r, carry2):
            for cc in range(H // 16):
                sl = pl.ds(cc * 16, 16)
                bufp[r, sl] = bufp[r, sl] + bufq[r, sl]
            return carry2

        lax.fori_loop(0, CHUNK, addrow, 0, unroll=2)
        pltpu.sync_copy(bufp, t_hbm.at[pl.ds(base, CHUNK)])
        return carry

    lax.fori_loop(0, NCHUNK, chunk, 0)


def _msg_body(t_ref, inv_ref, wc_ref, w2_ref, b2_ref, m_ref):
    hpre = t_ref[...] + jnp.dot(
        inv_ref[...], wc_ref[...], preferred_element_type=jnp.float32
    )
    h = _silu(hpre)
    m = jnp.dot(h, w2_ref[...], preferred_element_type=jnp.float32) + b2_ref[...]
    m_ref[...] = _silu(m)


def _msg(t, inv, wc, w2, b2):
    be = 4000
    grid = (N_EDGES // be,)
    return pl.pallas_call(
        _msg_body,
        grid=grid,
        in_specs=[
            pl.BlockSpec((be, H), lambda i: (i, 0)),
            pl.BlockSpec((be, D_INV), lambda i: (i, 0)),
            pl.BlockSpec((D_INV, H), lambda i: (0, 0)),
            pl.BlockSpec((H, H), lambda i: (0, 0)),
            pl.BlockSpec((1, H), lambda i: (0, 0)),
        ],
        out_specs=pl.BlockSpec((be, H), lambda i: (i, 0)),
        out_shape=jax.ShapeDtypeStruct((N_EDGES, H), jnp.float32),
    )(t, inv, wc, w2, b2)


@functools.cache
def _scatter_add_kernel():
    return pl.kernel(
        _scatter_add_body,
        out_type=jax.ShapeDtypeStruct((NC, N_NODES, H), jnp.float32),
        mesh=_sc_mesh(),
        scratch_types=[
            pltpu.VMEM((CHUNK,), jnp.int32),
            pltpu.VMEM((CHUNK, H), jnp.float32),
            pltpu.VMEM((ZROWS, H), jnp.float32),
            pltpu.VMEM_SHARED((N_NODES, H), jnp.float32),
            pltpu.SemaphoreType.DMA,
        ],
    )


def _scatter_add_body(m_hbm, dst_hbm, out_hbm, idx_d, mbuf, zbuf, agg_sh, sem):
    cid = lax.axis_index("c")
    sid = lax.axis_index("s")
    wid = sid * NC + cid
    wbase = wid * EPW

    zero = jnp.zeros((16,), jnp.float32)

    def zrow(r, carry):
        for cc in range(H // 16):
            zbuf[r, pl.ds(cc * 16, 16)] = zero
        return carry

    lax.fori_loop(0, ZROWS, zrow, 0)
    nstripe = jnp.where(sid == NS - 1, (N_NODES - (NS - 1) * STRIPE) // ZROWS, STRIPE // ZROWS)

    def zcopy(j, carry):
        pltpu.sync_copy(zbuf, agg_sh.at[pl.ds(pl.multiple_of(sid * STRIPE + j * ZROWS, 8), ZROWS)])
        return carry

    lax.fori_loop(0, nstripe, zcopy, 0)
    plsc.subcore_barrier()

    def chunk(c, carry):
        base = pl.multiple_of(wbase + c * CHUNK, 8)
        pltpu.sync_copy(dst_hbm.at[pl.ds(base, CHUNK)], idx_d)
        pltpu.sync_copy(m_hbm.at[pl.ds(base, CHUNK)], mbuf)
        pltpu.sync_copy(mbuf, agg_sh.at[idx_d], add=True)
        return carry

    lax.fori_loop(0, NCHUNK, chunk, 0)
    plsc.subcore_barrier()

    def ocopy(j, carry):
        r0 = pl.multiple_of(sid * STRIPE + j * ZROWS, 8)
        pltpu.sync_copy(agg_sh.at[pl.ds(r0, ZROWS)], zbuf)
        pltpu.sync_copy(zbuf, out_hbm.at[cid, pl.ds(r0, ZROWS)])
        return carry

    lax.fori_loop(0, nstripe, ocopy, 0)


def _upd_body(x_ref, agg_ref, wa_ref, wb_ref, b1_ref, w2_ref, b2_ref, o_ref):
    x = x_ref[...]
    agg = agg_ref[0] + agg_ref[1]
    u = _silu(
        jnp.dot(x, wa_ref[...], preferred_element_type=jnp.float32)
        + jnp.dot(agg, wb_ref[...], preferred_element_type=jnp.float32)
        + b1_ref[...]
    )
    o_ref[...] = x + jnp.dot(u, w2_ref[...], preferred_element_type=jnp.float32) + b2_ref[...]


def _upd(x, aggs, wa, wb, b1, w2, b2):
    bn = 2000
    grid = (N_NODES // bn,)
    return pl.pallas_call(
        _upd_body,
        grid=grid,
        in_specs=[
            pl.BlockSpec((bn, H), lambda i: (i, 0)),
            pl.BlockSpec((NC, bn, H), lambda i: (0, i, 0)),
            pl.BlockSpec((H, H), lambda i: (0, 0)),
            pl.BlockSpec((H, H), lambda i: (0, 0)),
            pl.BlockSpec((1, H), lambda i: (0, 0)),
            pl.BlockSpec((H, H), lambda i: (0, 0)),
            pl.BlockSpec((1, H), lambda i: (0, 0)),
        ],
        out_specs=pl.BlockSpec((bn, H), lambda i: (i, 0)),
        out_shape=jax.ShapeDtypeStruct((N_NODES, H), jnp.float32),
    )(x, aggs, wa, wb, b1, w2, b2)


def kernel(x, adj, inv, W_m1, b_m1, W_m2, b_m2, W_u1, b_u1, W_u2, b_u2):
    adj = adj.astype(jnp.int32)
    src = adj[0]
    dst = adj[1]

    p, q = _pq(x, W_m1[:H], W_m1[H : 2 * H], b_m1.reshape(1, H))
    t = _gather_add_kernel()(p, q, dst, src)
    m = _msg(t, inv, W_m1[2 * H :], W_m2, b_m2.reshape(1, H))
    aggs = _scatter_add_kernel()(m, dst)
    return _upd(
        x,
        aggs,
        W_u1[:H],
        W_u1[H:],
        b_u1.reshape(1, H),
        W_u2,
        b_u2.reshape(1, H),
    )


# speedup vs baseline: 3.5021x; 1.3941x over previous
import functools

import jax
import jax.numpy as jnp
from jax import lax
from jax.experimental import pallas as pl
from jax.experimental.pallas import tpu as pltpu
from jax.experimental.pallas import tpu_sc as plsc

N_NODES = 10000
N_EDGES = 320000
H = 128
D_INV = 16

NC = 2
NS = 16
NW = NC * NS

EPW = N_EDGES // NW
CHUNK = 80
NCHUNK = EPW // CHUNK
STRIPE = 640
ZROWS = 80

@functools.cache
def _sc_mesh():
    return plsc.VectorSubcoreMesh(
        core_axis_name="c", subcore_axis_name="s", num_cores=NC, num_subcores=NS
    )


def _silu(v):
    return v * (1.0 / (1.0 + jnp.exp(-v)))


def _pq_body(x_ref, wa_ref, wb_ref, b_ref, p_ref, q_ref):
    x = x_ref[...]
    p_ref[...] = jnp.dot(x, wa_ref[...], preferred_element_type=jnp.float32) + b_ref[...]
    q_ref[...] = jnp.dot(x, wb_ref[...], preferred_element_type=jnp.float32)


def _pq(x, wa, wb, b):
    bn = 2000
    grid = (N_NODES // bn,)
    return pl.pallas_call(
        _pq_body,
        grid=grid,
        in_specs=[
            pl.BlockSpec((bn, H), lambda i: (i, 0)),
            pl.BlockSpec((H, H), lambda i: (0, 0)),
            pl.BlockSpec((H, H), lambda i: (0, 0)),
            pl.BlockSpec((1, H), lambda i: (0, 0)),
        ],
        out_specs=[
            pl.BlockSpec((bn, H), lambda i: (i, 0)),
            pl.BlockSpec((bn, H), lambda i: (i, 0)),
        ],
        out_shape=[
            jax.ShapeDtypeStruct((N_NODES, H), jnp.float32),
            jax.ShapeDtypeStruct((N_NODES, H), jnp.float32),
        ],
    )(x, wa, wb, b)


@functools.cache
def _gather_add_kernel():
    return pl.kernel(
        _gather_add_body,
        out_type=jax.ShapeDtypeStruct((N_EDGES, H), jnp.float32),
        mesh=_sc_mesh(),
        scratch_types=[
            pltpu.VMEM((NCHUNK, CHUNK), jnp.int32),
            pltpu.VMEM((NCHUNK, CHUNK), jnp.int32),
            pltpu.VMEM((2, CHUNK, H), jnp.float32),
            pltpu.VMEM((2, CHUNK, H), jnp.float32),
            pltpu.SemaphoreType.DMA,
            pltpu.SemaphoreType.DMA,
            pltpu.SemaphoreType.DMA,
            pltpu.SemaphoreType.DMA,
        ],
    )


def _gather_add_body(
    p_hbm, q_hbm, dst_hbm, src_hbm, t_hbm, idx_d, idx_s, bufp, bufq, semga, semgb, semsa, semsb
):
    wid = lax.axis_index("s") * NC + lax.axis_index("c")
    wbase = wid * EPW
    pltpu.sync_copy(dst_hbm.at[wid], idx_d)
    pltpu.sync_copy(src_hbm.at[wid], idx_s)
    semg = (semga, semgb)
    sems = (semsa, semsb)

    def g_issue(c, slot):
        pltpu.async_copy(p_hbm.at[idx_d.at[c]], bufp.at[slot], semg[slot])
        pltpu.async_copy(q_hbm.at[idx_s.at[c]], bufq.at[slot], semg[slot])

    def g_wait(slot):
        pltpu.make_async_copy(p_hbm.at[idx_d.at[0]], bufp.at[slot], semg[slot]).wait()
        pltpu.make_async_copy(q_hbm.at[idx_s.at[0]], bufq.at[slot], semg[slot]).wait()

    def s_issue(c, slot):
        base = pl.multiple_of(wbase + c * CHUNK, 8)
        pltpu.async_copy(bufp.at[slot], t_hbm.at[pl.ds(base, CHUNK)], sems[slot])

    def s_wait(slot):
        pltpu.make_async_copy(bufp.at[slot], t_hbm.at[pl.ds(0, CHUNK)], sems[slot]).wait()

    def add(slot):
        def addrow(r, carry):
            for cc in range(H // 16):
                sl = pl.ds(cc * 16, 16)
                bufp[slot, r, sl] = bufp[slot, r, sl] + bufq[slot, r, sl]
            return carry

        lax.fori_loop(0, CHUNK, addrow, 0, unroll=2)

    g_issue(0, 0)

    def pair(i, carry):
        c0 = i * 2

        @pl.when(i > 0)
        def _():
            s_wait(1)

        g_issue(c0 + 1, 1)
        g_wait(0)
        add(0)
        s_issue(c0, 0)
        g_wait(1)
        add(1)
        s_wait(0)
        g_issue(c0 + 2, 0)
        s_issue(c0 + 1, 1)
        return carry

    lax.fori_loop(0, (NCHUNK - 1) // 2, pair, 0)
    s_wait(1)
    g_wait(0)
    add(0)
    s_issue(NCHUNK - 1, 0)
    s_wait(0)


def _msg_body(t_ref, inv_ref, wc_ref, w2_ref, b2_ref, m_ref):
    hpre = t_ref[...] + jnp.dot(
        inv_ref[...], wc_ref[...], preferred_element_type=jnp.float32
    )
    h = _silu(hpre)
    m = jnp.dot(h, w2_ref[...], preferred_element_type=jnp.float32) + b2_ref[...]
    m_ref[...] = _silu(m)


def _msg(t, inv, wc, w2, b2):
    be = 4000
    grid = (N_EDGES // be,)
    return pl.pallas_call(
        _msg_body,
        grid=grid,
        in_specs=[
            pl.BlockSpec((be, H), lambda i: (i, 0)),
            pl.BlockSpec((be, D_INV), lambda i: (i, 0)),
            pl.BlockSpec((D_INV, H), lambda i: (0, 0)),
            pl.BlockSpec((H, H), lambda i: (0, 0)),
            pl.BlockSpec((1, H), lambda i: (0, 0)),
        ],
        out_specs=pl.BlockSpec((be, H), lambda i: (i, 0)),
        out_shape=jax.ShapeDtypeStruct((N_EDGES, H), jnp.float32),
    )(t, inv, wc, w2, b2)


@functools.cache
def _scatter_add_kernel():
    return pl.kernel(
        _scatter_add_body,
        out_type=jax.ShapeDtypeStruct((NC, N_NODES, H), jnp.float32),
        mesh=_sc_mesh(),
        scratch_types=[
            pltpu.VMEM((NCHUNK, CHUNK), jnp.int32),
            pltpu.VMEM((2, CHUNK, H), jnp.float32),
            pltpu.VMEM((ZROWS, H), jnp.float32),
            pltpu.VMEM_SHARED((N_NODES, H), jnp.float32),
            pltpu.SemaphoreType.DMA,
            pltpu.SemaphoreType.DMA,
            pltpu.SemaphoreType.DMA,
            pltpu.SemaphoreType.DMA,
        ],
    )


def _scatter_add_body(m_hbm, dst_hbm, out_hbm, idx_d, mbuf, zbuf, agg_sh, semla, semlb, semaa, semab):
    cid = lax.axis_index("c")
    sid = lax.axis_index("s")
    wid = sid * NC + cid
    wbase = wid * EPW

    zero = jnp.zeros((16,), jnp.float32)

    def zrow(r, carry):
        for cc in range(H // 16):
            zbuf[r, pl.ds(cc * 16, 16)] = zero
        return carry

    lax.fori_loop(0, ZROWS, zrow, 0)
    nstripe = jnp.where(sid == NS - 1, (N_NODES - (NS - 1) * STRIPE) // ZROWS, STRIPE // ZROWS)

    def zcopy(j, carry):
        pltpu.sync_copy(zbuf, agg_sh.at[pl.ds(pl.multiple_of(sid * STRIPE + j * ZROWS, 8), ZROWS)])
        return carry

    lax.fori_loop(0, nstripe, zcopy, 0)
    pltpu.sync_copy(dst_hbm.at[wid], idx_d)
    plsc.subcore_barrier()

    seml = (semla, semlb)
    sema = (semaa, semab)

    def m_issue(c, slot):
        base = pl.multiple_of(wbase + c * CHUNK, 8)
        pltpu.async_copy(m_hbm.at[pl.ds(base, CHUNK)], mbuf.at[slot], seml[slot])

    def m_wait(slot):
        pltpu.make_async_copy(m_hbm.at[pl.ds(0, CHUNK)], mbuf.at[slot], seml[slot]).wait()

    def a_issue(c, slot):
        pltpu.async_copy(mbuf.at[slot], agg_sh.at[idx_d.at[c]], sema[slot], add=True)

    def a_wait(slot):
        pltpu.make_async_copy(mbuf.at[slot], agg_sh.at[idx_d.at[0]], sema[slot]).wait()

    m_issue(0, 0)

    def pair(i, carry):
        c0 = i * 2

        @pl.when(i > 0)
        def _():
            a_wait(1)

        m_issue(c0 + 1, 1)
        m_wait(0)
        a_issue(c0, 0)
        m_wait(1)
        a_wait(0)
        m_issue(c0 + 2, 0)
        a_issue(c0 + 1, 1)
        return carry

    lax.fori_loop(0, (NCHUNK - 1) // 2, pair, 0)
    a_wait(1)
    m_wait(0)
    a_issue(NCHUNK - 1, 0)
    a_wait(0)
    plsc.subcore_barrier()

    def ocopy(j, carry):
        r0 = pl.multiple_of(sid * STRIPE + j * ZROWS, 8)
        pltpu.sync_copy(agg_sh.at[pl.ds(r0, ZROWS)], zbuf)
        pltpu.sync_copy(zbuf, out_hbm.at[cid, pl.ds(r0, ZROWS)])
        return carry

    lax.fori_loop(0, nstripe, ocopy, 0)


def _upd_body(x_ref, agg_ref, wa_ref, wb_ref, b1_ref, w2_ref, b2_ref, o_ref):
    x = x_ref[...]
    agg = agg_ref[0] + agg_ref[1]
    u = _silu(
        jnp.dot(x, wa_ref[...], preferred_element_type=jnp.float32)
        + jnp.dot(agg, wb_ref[...], preferred_element_type=jnp.float32)
        + b1_ref[...]
    )
    o_ref[...] = x + jnp.dot(u, w2_ref[...], preferred_element_type=jnp.float32) + b2_ref[...]


def _upd(x, aggs, wa, wb, b1, w2, b2):
    bn = 2000
    grid = (N_NODES // bn,)
    return pl.pallas_call(
        _upd_body,
        grid=grid,
        in_specs=[
            pl.BlockSpec((bn, H), lambda i: (i, 0)),
            pl.BlockSpec((NC, bn, H), lambda i: (0, i, 0)),
            pl.BlockSpec((H, H), lambda i: (0, 0)),
            pl.BlockSpec((H, H), lambda i: (0, 0)),
            pl.BlockSpec((1, H), lambda i: (0, 0)),
            pl.BlockSpec((H, H), lambda i: (0, 0)),
            pl.BlockSpec((1, H), lambda i: (0, 0)),
        ],
        out_specs=pl.BlockSpec((bn, H), lambda i: (i, 0)),
        out_shape=jax.ShapeDtypeStruct((N_NODES, H), jnp.float32),
    )(x, aggs, wa, wb, b1, w2, b2)


def kernel(x, adj, inv, W_m1, b_m1, W_m2, b_m2, W_u1, b_u1, W_u2, b_u2):
    adj = adj.astype(jnp.int32)
    src = adj[0].reshape(NW, NCHUNK, CHUNK)
    dst = adj[1].reshape(NW, NCHUNK, CHUNK)

    p, q = _pq(x, W_m1[:H], W_m1[H : 2 * H], b_m1.reshape(1, H))
    t = _gather_add_kernel()(p, q, dst, src)
    m = _msg(t, inv, W_m1[2 * H :], W_m2, b_m2.reshape(1, H))
    aggs = _scatter_add_kernel()(m, dst)
    return _upd(
        x,
        aggs,
        W_u1[:H],
        W_u1[H:],
        b_u1.reshape(1, H),
        W_u2,
        b_u2.reshape(1, H),
    )


# speedup vs baseline: 4.4969x; 1.2841x over previous
import functools

import jax
import jax.numpy as jnp
from jax import lax
from jax.experimental import pallas as pl
from jax.experimental.pallas import tpu as pltpu
from jax.experimental.pallas import tpu_sc as plsc

N_NODES = 10000
N_EDGES = 320000
H = 128
D_INV = 16

NC = 2
NS = 16
NW = NC * NS

EPW = N_EDGES // NW
CHUNK = 80
NCHUNK = EPW // CHUNK
STRIPE = 640
ZROWS = 80

@functools.cache
def _sc_mesh():
    return plsc.VectorSubcoreMesh(
        core_axis_name="c", subcore_axis_name="s", num_cores=NC, num_subcores=NS
    )


def _silu(v):
    return v * (1.0 / (1.0 + jnp.exp(-v)))


def _pq_body(x_ref, wa_ref, wb_ref, b_ref, p_ref, q_ref):
    x = x_ref[...]
    p_ref[...] = jnp.dot(x, wa_ref[...], preferred_element_type=jnp.float32) + b_ref[...]
    q_ref[...] = jnp.dot(x, wb_ref[...], preferred_element_type=jnp.float32)


def _pq(x, wa, wb, b):
    bn = 2000
    grid = (N_NODES // bn,)
    return pl.pallas_call(
        _pq_body,
        grid=grid,
        in_specs=[
            pl.BlockSpec((bn, H), lambda i: (i, 0)),
            pl.BlockSpec((H, H), lambda i: (0, 0)),
            pl.BlockSpec((H, H), lambda i: (0, 0)),
            pl.BlockSpec((1, H), lambda i: (0, 0)),
        ],
        out_specs=[
            pl.BlockSpec((bn, H), lambda i: (i, 0)),
            pl.BlockSpec((bn, H), lambda i: (i, 0)),
        ],
        out_shape=[
            jax.ShapeDtypeStruct((N_NODES, H), jnp.float32),
            jax.ShapeDtypeStruct((N_NODES, H), jnp.float32),
        ],
    )(x, wa, wb, b)


@functools.cache
def _gather_add_kernel():
    return pl.kernel(
        _gather_add_body,
        out_type=jax.ShapeDtypeStruct((N_EDGES, H), jnp.float32),
        mesh=_sc_mesh(),
        scratch_types=[
            pltpu.VMEM((NCHUNK, CHUNK), jnp.int32),
            pltpu.VMEM((NCHUNK, CHUNK), jnp.int32),
            pltpu.VMEM((3, CHUNK, H), jnp.float32),
            pltpu.VMEM((3, CHUNK, H), jnp.float32),
            pltpu.VMEM((2, CHUNK, H), jnp.float32),
            pltpu.SemaphoreType.DMA,
            pltpu.SemaphoreType.DMA,
            pltpu.SemaphoreType.DMA,
            pltpu.SemaphoreType.DMA,
            pltpu.SemaphoreType.DMA,
        ],
    )


def _gather_add_body(
    p_hbm, q_hbm, dst_hbm, src_hbm, t_hbm,
    idx_d, idx_s, bufp, bufq, bufo, semg0, semg1, semg2, sems0, sems1,
):
    wid = lax.axis_index("s") * NC + lax.axis_index("c")
    wbase = wid * EPW
    pltpu.sync_copy(dst_hbm.at[wid], idx_d)
    pltpu.sync_copy(src_hbm.at[wid], idx_s)
    semg = (semg0, semg1, semg2)
    sems = (sems0, sems1)

    def g_issue(c, slot):
        pltpu.async_copy(p_hbm.at[idx_d.at[c]], bufp.at[slot], semg[slot])
        pltpu.async_copy(q_hbm.at[idx_s.at[c]], bufq.at[slot], semg[slot])

    def g_wait(slot):
        pltpu.make_async_copy(p_hbm.at[idx_d.at[0]], bufp.at[slot], semg[slot]).wait()
        pltpu.make_async_copy(q_hbm.at[idx_s.at[0]], bufq.at[slot], semg[slot]).wait()

    def s_issue(c, oslot):
        base = pl.multiple_of(wbase + c * CHUNK, 8)
        pltpu.async_copy(bufo.at[oslot], t_hbm.at[pl.ds(base, CHUNK)], sems[oslot])

    def s_wait(oslot):
        pltpu.make_async_copy(bufo.at[oslot], t_hbm.at[pl.ds(0, CHUNK)], sems[oslot]).wait()

    def add(slot, oslot):
        def addrow(r, carry):
            for cc in range(H // 16):
                sl = pl.ds(cc * 16, 16)
                bufo[oslot, r, sl] = bufp[slot, r, sl] + bufq[slot, r, sl]
            return carry

        lax.fori_loop(0, CHUNK, addrow, 0, unroll=2)

    def substep(c, slot, oslot, issue_next, wait_store):
        if issue_next:
            g_issue(c + 2, (slot + 2) % 3)
        g_wait(slot)
        if wait_store is None:
            s_wait(oslot)
        elif wait_store is not None and wait_store is not False:
            @pl.when(wait_store)
            def _():
                s_wait(oslot)
        add(slot, oslot)
        s_issue(c, oslot)

    g_issue(0, 0)
    g_issue(1, 1)

    def group(i, carry):
        c0 = i * 6
        for j in range(6):
            ws = (i > 0) if j < 2 else None
            substep(c0 + j, j % 3, j % 2, True, ws)
        return carry

    lax.fori_loop(0, (NCHUNK - 5) // 6, group, 0)
    for c in range(NCHUNK - 5, NCHUNK):
        substep(c, c % 3, c % 2, c + 2 < NCHUNK, None)
    s_wait((NCHUNK - 2) % 2)
    s_wait((NCHUNK - 1) % 2)


def _msg_body(t_ref, inv_ref, wc_ref, w2_ref, b2_ref, m_ref):
    hpre = t_ref[...] + jnp.dot(
        inv_ref[...], wc_ref[...], preferred_element_type=jnp.float32
    )
    h = _silu(hpre)
    m = jnp.dot(h, w2_ref[...], preferred_element_type=jnp.float32) + b2_ref[...]
    m_ref[...] = _silu(m)


def _msg(t, inv, wc, w2, b2):
    be = 4000
    grid = (N_EDGES // be,)
    return pl.pallas_call(
        _msg_body,
        grid=grid,
        in_specs=[
            pl.BlockSpec((be, H), lambda i: (i, 0)),
            pl.BlockSpec((be, D_INV), lambda i: (i, 0)),
            pl.BlockSpec((D_INV, H), lambda i: (0, 0)),
            pl.BlockSpec((H, H), lambda i: (0, 0)),
            pl.BlockSpec((1, H), lambda i: (0, 0)),
        ],
        out_specs=pl.BlockSpec((be, H), lambda i: (i, 0)),
        out_shape=jax.ShapeDtypeStruct((N_EDGES, H), jnp.float32),
    )(t, inv, wc, w2, b2)


@functools.cache
def _scatter_add_kernel():
    return pl.kernel(
        _scatter_add_body,
        out_type=jax.ShapeDtypeStruct((NC, N_NODES, H), jnp.float32),
        mesh=_sc_mesh(),
        scratch_types=[
            pltpu.VMEM((NCHUNK, CHUNK), jnp.int32),
            pltpu.VMEM((2, CHUNK, H), jnp.float32),
            pltpu.VMEM((ZROWS, H), jnp.float32),
            pltpu.VMEM_SHARED((N_NODES, H), jnp.float32),
            pltpu.SemaphoreType.DMA,
            pltpu.SemaphoreType.DMA,
            pltpu.SemaphoreType.DMA,
            pltpu.SemaphoreType.DMA,
        ],
    )


def _scatter_add_body(m_hbm, dst_hbm, out_hbm, idx_d, mbuf, zbuf, agg_sh, semla, semlb, semaa, semab):
    cid = lax.axis_index("c")
    sid = lax.axis_index("s")
    wid = sid * NC + cid
    wbase = wid * EPW

    zero = jnp.zeros((16,), jnp.float32)

    def zrow(r, carry):
        for cc in range(H // 16):
            zbuf[r, pl.ds(cc * 16, 16)] = zero
        return carry

    lax.fori_loop(0, ZROWS, zrow, 0)
    nstripe = jnp.where(sid == NS - 1, (N_NODES - (NS - 1) * STRIPE) // ZROWS, STRIPE // ZROWS)

    def zcopy(j, carry):
        pltpu.sync_copy(zbuf, agg_sh.at[pl.ds(pl.multiple_of(sid * STRIPE + j * ZROWS, 8), ZROWS)])
        return carry

    lax.fori_loop(0, nstripe, zcopy, 0)
    pltpu.sync_copy(dst_hbm.at[wid], idx_d)
    plsc.subcore_barrier()

    seml = (semla, semlb)
    sema = (semaa, semab)

    def m_issue(c, slot):
        base = pl.multiple_of(wbase + c * CHUNK, 8)
        pltpu.async_copy(m_hbm.at[pl.ds(base, CHUNK)], mbuf.at[slot], seml[slot])

    def m_wait(slot):
        pltpu.make_async_copy(m_hbm.at[pl.ds(0, CHUNK)], mbuf.at[slot], seml[slot]).wait()

    def a_issue(c, slot):
        pltpu.async_copy(mbuf.at[slot], agg_sh.at[idx_d.at[c]], sema[slot], add=True)

    def a_wait(slot):
        pltpu.make_async_copy(mbuf.at[slot], agg_sh.at[idx_d.at[0]], sema[slot]).wait()

    m_issue(0, 0)

    def pair(i, carry):
        c0 = i * 2

        @pl.when(i > 0)
        def _():
            a_wait(1)

        m_issue(c0 + 1, 1)
        m_wait(0)
        a_issue(c0, 0)
        m_wait(1)
        a_wait(0)
        m_issue(c0 + 2, 0)
        a_issue(c0 + 1, 1)
        return carry

    lax.fori_loop(0, (NCHUNK - 1) // 2, pair, 0)
    a_wait(1)
    m_wait(0)
    a_issue(NCHUNK - 1, 0)
    a_wait(0)
    plsc.subcore_barrier()

    def ocopy(j, carry):
        r0 = pl.multiple_of(sid * STRIPE + j * ZROWS, 8)
        pltpu.sync_copy(agg_sh.at[pl.ds(r0, ZROWS)], zbuf)
        pltpu.sync_copy(zbuf, out_hbm.at[cid, pl.ds(r0, ZROWS)])
        return carry

    lax.fori_loop(0, nstripe, ocopy, 0)


def _upd_body(x_ref, agg_ref, wa_ref, wb_ref, b1_ref, w2_ref, b2_ref, o_ref):
    x = x_ref[...]
    agg = agg_ref[0] + agg_ref[1]
    u = _silu(
        jnp.dot(x, wa_ref[...], preferred_element_type=jnp.float32)
        + jnp.dot(agg, wb_ref[...], preferred_element_type=jnp.float32)
        + b1_ref[...]
    )
    o_ref[...] = x + jnp.dot(u, w2_ref[...], preferred_element_type=jnp.float32) + b2_ref[...]


def _upd(x, aggs, wa, wb, b1, w2, b2):
    bn = 2000
    grid = (N_NODES // bn,)
    return pl.pallas_call(
        _upd_body,
        grid=grid,
        in_specs=[
            pl.BlockSpec((bn, H), lambda i: (i, 0)),
            pl.BlockSpec((NC, bn, H), lambda i: (0, i, 0)),
            pl.BlockSpec((H, H), lambda i: (0, 0)),
            pl.BlockSpec((H, H), lambda i: (0, 0)),
            pl.BlockSpec((1, H), lambda i: (0, 0)),
            pl.BlockSpec((H, H), lambda i: (0, 0)),
            pl.BlockSpec((1, H), lambda i: (0, 0)),
        ],
        out_specs=pl.BlockSpec((bn, H), lambda i: (i, 0)),
        out_shape=jax.ShapeDtypeStruct((N_NODES, H), jnp.float32),
    )(x, aggs, wa, wb, b1, w2, b2)


def kernel(x, adj, inv, W_m1, b_m1, W_m2, b_m2, W_u1, b_u1, W_u2, b_u2):
    adj = adj.astype(jnp.int32)
    src = adj[0].reshape(NW, NCHUNK, CHUNK)
    dst = adj[1].reshape(NW, NCHUNK, CHUNK)

    p, q = _pq(x, W_m1[:H], W_m1[H : 2 * H], b_m1.reshape(1, H))
    t = _gather_add_kernel()(p, q, dst, src)
    m = _msg(t, inv, W_m1[2 * H :], W_m2, b_m2.reshape(1, H))
    aggs = _scatter_add_kernel()(m, dst)
    return _upd(
        x,
        aggs,
        W_u1[:H],
        W_u1[H:],
        b_u1.reshape(1, H),
        W_u2,
        b_u2.reshape(1, H),
    )
